# baseline (speedup 1.0000x reference)
import functools

import jax
import jax.numpy as jnp
from jax import lax
from jax.experimental import pallas as pl
from jax.experimental.pallas import tpu as pltpu
from jax.experimental.pallas import tpu_sc as plsc

TOPK = 8
CAP_FACTOR = 1.25
LB_W = 0.01
TB = 1024
FLOAT_BS_ITERS = 48


def _route_body(x_ref, w_ref, probs_ref, disp_ref, psum_ref, tpe_ref,
                loss_ref):
    num_e = w_ref.shape[0]
    x = x_ref[...]
    w = w_ref[...]
    logits = jax.lax.dot_general(
        x, w, (((1,), (1,)), ((), ())), preferred_element_type=jnp.float32)
    mx = jnp.max(logits, axis=-1, keepdims=True)
    ex = jnp.exp(logits - mx)
    p = ex / jnp.sum(ex, axis=-1, keepdims=True)
    probs_ref[...] = p

    lane = jax.lax.broadcasted_iota(jnp.int32, p.shape, 1)
    work = p
    ssum = jnp.zeros((p.shape[0], 1), jnp.float32)
    for _ in range(TOPK):
        m = jnp.max(work, axis=-1, keepdims=True)
        cand = jnp.where(work == m, lane, num_e)
        sel = jnp.min(cand, axis=-1, keepdims=True)
        work = jnp.where(lane == sel, -1.0, work)
        ssum = ssum + m
    d = jnp.where(work < 0.0, p, 0.0) / ssum
    disp_ref[...] = d

    part_p = jnp.sum(p, axis=0, keepdims=True)
    part_t = jnp.sum(d, axis=0, keepdims=True)
    i = pl.program_id(0)

    @pl.when(i == 0)
    def _():
        psum_ref[...] = part_p
        tpe_ref[...] = part_t

    @pl.when(i != 0)
    def _():
        psum_ref[...] += part_p
        tpe_ref[...] += part_t

    @pl.when(i == pl.num_programs(0) - 1)
    def _():
        n_tok = pl.num_programs(0) * x_ref.shape[0]
        loss = jnp.sum(psum_ref[...] * tpe_ref[...]) * jnp.float32(
            LB_W / n_tok)
        loss_ref[...] = loss.reshape(1, 1)


def _route_call(x, router_w, n_tok, d, num_e):
    return pl.pallas_call(
        _route_body,
        grid=(n_tok // TB,),
        in_specs=[
            pl.BlockSpec((TB, d), lambda i: (i, 0)),
            pl.BlockSpec((num_e, d), lambda i: (0, 0)),
        ],
        out_specs=[
            pl.BlockSpec((TB, num_e), lambda i: (i, 0)),
            pl.BlockSpec((TB, num_e), lambda i: (i, 0)),
            pl.BlockSpec((1, num_e), lambda i: (0, 0)),
            pl.BlockSpec((1, num_e), lambda i: (0, 0)),
            pl.BlockSpec((1, 1), lambda i: (0, 0)),
        ],
        out_shape=[
            jax.ShapeDtypeStruct((n_tok, num_e), jnp.float32),
            jax.ShapeDtypeStruct((n_tok, num_e), jnp.float32),
            jax.ShapeDtypeStruct((1, num_e), jnp.float32),
            jax.ShapeDtypeStruct((1, num_e), jnp.float32),
            jax.ShapeDtypeStruct((1, 1), jnp.float32),
        ],
    )(x, router_w)


def _t_body(x_ref, o_ref):
    o_ref[...] = jnp.transpose(x_ref[...], (1, 0))


def _transpose_to_expert_major(disp, n_tok, num_e):
    return pl.pallas_call(
        _t_body,
        grid=(n_tok // TB,),
        in_specs=[pl.BlockSpec((TB, num_e), lambda i: (i, 0))],
        out_specs=pl.BlockSpec((num_e, TB), lambda i: (0, i)),
        out_shape=jax.ShapeDtypeStruct((num_e, n_tok), jnp.float32),
    )(disp)


def _transpose_to_token_major(masked_t, n_tok, num_e):
    return pl.pallas_call(
        _t_body,
        grid=(n_tok // TB,),
        in_specs=[pl.BlockSpec((num_e, TB), lambda i: (0, i))],
        out_specs=pl.BlockSpec((TB, num_e), lambda i: (i, 0)),
        out_shape=jax.ShapeDtypeStruct((n_tok, num_e), jnp.float32),
    )(masked_t)


def _sc_cap_body(dispt_hbm, tpe_hbm, out_hbm, col_ref, tpe16_ref, tmp_ref,
                 *, capacity, n_tok, num_e):
    wid = lax.axis_index("s") * 2 + lax.axis_index("c")
    e_per = num_e // 32
    nv = n_tok // 16
    lanes = lax.iota(jnp.int32, 16)
    ones16 = jnp.full((16,), 1, jnp.int32)
    zeros16 = jnp.zeros((16,), jnp.int32)

    def _splat_sum(vec):
        for sh in (8, 4, 2, 1):
            tmp_ref[pl.ds(0, 16)] = vec
            tmp_ref[pl.ds(16, 16)] = vec
            vec = vec + tmp_ref[pl.ds(sh, 16)]
        return vec

    for k in range(e_per):
        e = wid * e_per + k
        pltpu.sync_copy(dispt_hbm.at[e], col_ref)
        pltpu.sync_copy(tpe_hbm.at[e], tpe16_ref)
        no_drop = tpe16_ref[...] <= jnp.float32(capacity)

        def _load(j):
            return col_ref[pl.ds(j * 16, 16)]

        def _cnt_ge(t_vec):
            def body(j, acc):
                return acc + jnp.where(_load(j) >= t_vec, ones16, zeros16)

            return _splat_sum(lax.fori_loop(0, nv, body, zeros16))

        def _cnt_gt(t_vec):
            def body(j, acc):
                return acc + jnp.where(_load(j) > t_vec, ones16, zeros16)

            return _splat_sum(lax.fori_loop(0, nv, body, zeros16))

        cap_vec = jnp.full((16,), capacity, jnp.int32)
        half = jnp.full((16,), 0.5, jnp.float32)

        def _bs(_, lh):
            lo, hi = lh
            mid = (lo + hi) * half
            ge = _cnt_ge(mid) >= cap_vec
            return jnp.where(ge, mid, lo), jnp.where(ge, hi, mid)

        vstar, _ = lax.fori_loop(
            0, FLOAT_BS_ITERS, _bs,
            (jnp.zeros((16,), jnp.float32),
             jnp.full((16,), 1.001, jnp.float32)))

        n_eq = cap_vec - _cnt_gt(vstar)

        def _cnt_eq_le(i_vec):
            def body(j, acc):
                hit = jnp.logical_and(_load(j) == vstar,
                                      lanes + j * 16 <= i_vec)
                return acc + jnp.where(hit, ones16, zeros16)

            return _splat_sum(lax.fori_loop(0, nv, body, zeros16))

        def _bsi(_, lh):
            lo, hi = lh
            mid = (lo + hi) >> 1
            ok = _cnt_eq_le(mid) >= n_eq
            return jnp.where(ok, lo, mid), jnp.where(ok, mid, hi)

        _, istar = lax.fori_loop(
            0, 15, _bsi, (jnp.full((16,), -1, jnp.int32),
                          jnp.full((16,), n_tok - 1, jnp.int32)))

        def _apply(j, carry):
            v = _load(j)
            idx = lanes + j * 16
            keep = jnp.logical_or(
                v > vstar, jnp.logical_and(v == vstar, idx <= istar))
            keep = jnp.logical_or(keep, no_drop)
            col_ref[pl.ds(j * 16, 16)] = jnp.where(
                keep, v, jnp.zeros((16,), jnp.float32))
            return carry

        lax.fori_loop(0, nv, _apply, jnp.int32(0))
        pltpu.sync_copy(col_ref, out_hbm.at[e])


def _sc_cap_call(disp_t, tpe_rows, n_tok, num_e, capacity):
    mesh = plsc.VectorSubcoreMesh(core_axis_name="c", subcore_axis_name="s")
    fn = pl.kernel(
        functools.partial(_sc_cap_body, capacity=capacity, n_tok=n_tok,
                          num_e=num_e),
        mesh=mesh,
        out_type=jax.ShapeDtypeStruct((num_e, n_tok), jnp.float32),
        scratch_types=[pltpu.VMEM((n_tok,), jnp.float32),
                       pltpu.VMEM((16,), jnp.float32),
                       pltpu.VMEM((32,), jnp.int32)],
    )
    return fn(disp_t, tpe_rows)


def kernel(hidden_states, router_w):
    b, s, d = hidden_states.shape
    num_e = router_w.shape[0]
    n_tok = b * s
    capacity = int(CAP_FACTOR * s * b / num_e)
    x = hidden_states.reshape(n_tok, d)

    probs, disp, _psum, tpe, loss = _route_call(x, router_w, n_tok, d, num_e)

    def _drop_path(dm, t):
        disp_t = _transpose_to_expert_major(dm, n_tok, num_e)
        tpe_rows = jnp.broadcast_to(t.reshape(num_e, 1), (num_e, 16))
        masked_t = _sc_cap_call(disp_t, tpe_rows, n_tok, num_e, capacity)
        return _transpose_to_token_major(masked_t, n_tok, num_e)

    any_over = jnp.any(tpe > jnp.float32(capacity))
    dropped = jax.lax.cond(any_over, _drop_path, lambda dm, t: dm, disp, tpe)

    d_out = dropped.reshape(b, s, num_e)
    return d_out, d_out, loss.reshape(()), probs.reshape(b, s, num_e)
